# Initial kernel scaffold; baseline (speedup 1.0000x reference)
import functools

import jax
import jax.numpy as jnp
from jax import lax
from jax.experimental import pallas as pl
from jax.experimental.pallas import tpu as pltpu
from jax.experimental.pallas import tpu_sc as plsc

N_NODES = 10000
N_EDGES = 160000
D_FEAT = 256
HALF = 128
N_CLASSES = 2
NC = 2
NS = 16
L = 16

CHUNK = 128
CHUNKS_PER_TILE = 79
E_TILE = CHUNK * CHUNKS_PER_TILE
E_PAD = E_TILE * NS
IDX_ROWS = E_PAD // CHUNK
TRASH = 16
ACC_ROWS = N_NODES + TRASH

E_W = E_PAD // (NC * NS)
BLK_W = E_W // L
DEG_ROWS = 640

_mesh = plsc.VectorSubcoreMesh(core_axis_name="c", subcore_axis_name="s")


@functools.partial(
    pl.kernel,
    out_type=jax.ShapeDtypeStruct((NC, DEG_ROWS, L), jnp.float32),
    mesh=_mesh,
    scratch_types=[
        pltpu.VMEM((E_W,), jnp.int32),
        pltpu.VMEM((DEG_ROWS, L), jnp.float32),
        pltpu.VMEM((5, CHUNK), jnp.int32),
        pltpu.VMEM((40, L), jnp.float32),
        pltpu.VMEM_SHARED((DEG_ROWS, L), jnp.float32),
    ],
)
def _deg_kernel(dstp_hbm, iota_hbm, out_hbm, idx_v, hist, iota_v, zrow, deg_sh):
    c = lax.axis_index("c")
    s = lax.axis_index("s")
    w = c * NS + s

    zeros16 = jnp.zeros((L,), jnp.float32)

    @pl.loop(0, DEG_ROWS)
    def _(i):
        hist[i, :] = zeros16

    @pl.loop(0, 40)
    def _(i):
        zrow[i, :] = zeros16

    pltpu.sync_copy(zrow, deg_sh.at[pl.ds(s * 40, 40)])
    pltpu.sync_copy(iota_hbm, iota_v)
    pltpu.sync_copy(dstp_hbm.at[pl.ds(w * E_W, E_W)], idx_v)
    plsc.subcore_barrier()

    ones16 = jnp.ones((L,), jnp.float32)

    @pl.loop(0, BLK_W)
    def _(j):
        d = idx_v[pl.ds(j * L, L)]
        rows = lax.shift_right_logical(d, 4)
        cols = lax.bitwise_and(d, 15)
        plsc.addupdate_scatter(hist, [rows, cols], ones16)

    for r in range(5):
        pltpu.sync_copy(
            hist.at[pl.ds(r * CHUNK, CHUNK)],
            deg_sh.at[iota_v.at[r]],
            add=True,
        )
    plsc.subcore_barrier()
    pltpu.sync_copy(deg_sh.at[pl.ds(s * 40, 40)], out_hbm.at[c, pl.ds(s * 40, 40)])


def _mm_body(x_ref, w_ref, degt_ref, z_ref, dis_ref):
    xw = jnp.dot(x_ref[...], w_ref[...], preferred_element_type=jnp.float32)
    deg = degt_ref[:, 0:1] + degt_ref[:, 1:2] + 1.0
    dis = lax.rsqrt(deg)
    z_ref[0, :, :] = xw[:, :HALF] * dis
    z_ref[1, :, :] = xw[:, HALF:] * dis
    dis_ref[...] = dis


_ROWS_BLK = 1000


def _mm(x, w1, degt):
    return pl.pallas_call(
        _mm_body,
        grid=(N_NODES // _ROWS_BLK,),
        in_specs=[
            pl.BlockSpec((_ROWS_BLK, D_FEAT), lambda i: (i, 0)),
            pl.BlockSpec((D_FEAT, D_FEAT), lambda i: (0, 0)),
            pl.BlockSpec((_ROWS_BLK, 2), lambda i: (i, 0)),
        ],
        out_specs=[
            pl.BlockSpec((NC, _ROWS_BLK, HALF), lambda i: (0, i, 0)),
            pl.BlockSpec((_ROWS_BLK, 1), lambda i: (i, 0)),
        ],
        out_shape=[
            jax.ShapeDtypeStruct((NC, N_NODES, HALF), jnp.float32),
            jax.ShapeDtypeStruct((N_NODES, 1), jnp.float32),
        ],
    )(x, w1, degt)


@functools.partial(
    pl.kernel,
    out_type=jax.ShapeDtypeStruct((NC, N_NODES, HALF), jnp.float32),
    mesh=_mesh,
    scratch_types=[
        pltpu.VMEM((CHUNKS_PER_TILE, CHUNK), jnp.int32),
        pltpu.VMEM((CHUNKS_PER_TILE, CHUNK), jnp.int32),
        pltpu.VMEM((CHUNK, HALF), jnp.float32),
        pltpu.VMEM_SHARED((ACC_ROWS, HALF), jnp.float32),
    ],
)
def _agg_kernel(z_hbm, srcp_hbm, dstp_hbm, out_hbm, src_v, dst_v, buf, acc_sh):
    c = lax.axis_index("c")
    s = lax.axis_index("s")

    zeros16 = jnp.zeros((L,), jnp.float32)

    @pl.loop(0, CHUNK)
    def _(i):
        for k in range(HALF // L):
            buf[i, pl.ds(k * L, L)] = zeros16

    base = s * 626
    for off, n in ((0, 128), (128, 128), (256, 128), (384, 128), (512, 114)):
        pltpu.sync_copy(buf.at[pl.ds(0, n)], acc_sh.at[pl.ds(base + off, n)])

    pltpu.sync_copy(srcp_hbm.at[pl.ds(s * CHUNKS_PER_TILE, CHUNKS_PER_TILE)], src_v)
    pltpu.sync_copy(dstp_hbm.at[pl.ds(s * CHUNKS_PER_TILE, CHUNKS_PER_TILE)], dst_v)
    plsc.subcore_barrier()

    zc = z_hbm.at[c]

    @pl.loop(0, CHUNKS_PER_TILE)
    def _(j):
        pltpu.sync_copy(zc.at[src_v.at[j]], buf)
        pltpu.sync_copy(buf, acc_sh.at[dst_v.at[j]], add=True)

    plsc.subcore_barrier()
    rows = s * 625
    pltpu.sync_copy(
        acc_sh.at[pl.ds(rows, 625)], out_hbm.at[c, pl.ds(rows, 625)]
    )


def _fin_body(acc_ref, z_ref, dis_ref, b1_ref, wfc_ref, bfc_ref, out_ref, s_acc):
    i = pl.program_id(0)
    dis = dis_ref[...]
    h0 = jnp.maximum(dis * (acc_ref[0, :, :] + z_ref[0, :, :]) + b1_ref[0:1, :], 0.0)
    h1 = jnp.maximum(dis * (acc_ref[1, :, :] + z_ref[1, :, :]) + b1_ref[1:2, :], 0.0)
    p0 = jnp.sum(h0, axis=0, keepdims=True)
    p1 = jnp.sum(h1, axis=0, keepdims=True)

    @pl.when(i == 0)
    def _():
        s_acc[0:1, :] = p0
        s_acc[1:2, :] = p1

    @pl.when(i > 0)
    def _():
        s_acc[0:1, :] = s_acc[0:1, :] + p0
        s_acc[1:2, :] = s_acc[1:2, :] + p1

    @pl.when(i == pl.num_programs(0) - 1)
    def _():
        h = s_acc[...] * (1.0 / N_NODES)
        logits = (
            jnp.dot(h[0:1, :], wfc_ref[pl.ds(0, HALF), :],
                    preferred_element_type=jnp.float32)
            + jnp.dot(h[1:2, :], wfc_ref[pl.ds(HALF, HALF), :],
                      preferred_element_type=jnp.float32)
            + bfc_ref[...]
        )
        m = jnp.max(logits, axis=1, keepdims=True)
        lse = m + jnp.log(jnp.sum(jnp.exp(logits - m), axis=1, keepdims=True))
        out_ref[...] = logits - lse


def _fin(acc, z, dis, b1r, wfc, bfcr):
    return pl.pallas_call(
        _fin_body,
        grid=(N_NODES // _ROWS_BLK,),
        in_specs=[
            pl.BlockSpec((NC, _ROWS_BLK, HALF), lambda i: (0, i, 0)),
            pl.BlockSpec((NC, _ROWS_BLK, HALF), lambda i: (0, i, 0)),
            pl.BlockSpec((_ROWS_BLK, 1), lambda i: (i, 0)),
            pl.BlockSpec((NC, HALF), lambda i: (0, 0)),
            pl.BlockSpec((D_FEAT, N_CLASSES), lambda i: (0, 0)),
            pl.BlockSpec((1, N_CLASSES), lambda i: (0, 0)),
        ],
        out_specs=pl.BlockSpec((1, N_CLASSES), lambda i: (0, 0)),
        out_shape=jax.ShapeDtypeStruct((1, N_CLASSES), jnp.float32),
        scratch_shapes=[pltpu.VMEM((NC, HALF), jnp.float32)],
    )(acc, z, dis, b1r, wfc, bfcr)


def kernel(x, edge_index, W1, b1, W_fc, b_fc):
    src = edge_index[0]
    dst = edge_index[1]
    pad = E_PAD - N_EDGES
    srcp = jnp.concatenate([src, jnp.zeros((pad,), jnp.int32)])
    dstp = jnp.concatenate([dst, jnp.full((pad,), N_NODES, jnp.int32)])
    iota5 = jnp.arange(DEG_ROWS, dtype=jnp.int32).reshape(5, CHUNK)

    degp = _deg_kernel(dstp, iota5)
    degt = degp.reshape(NC, DEG_ROWS * L)[:, :N_NODES].T
    z, dis = _mm(x, W1, degt)
    acc = _agg_kernel(z, srcp.reshape(IDX_ROWS, CHUNK),
                      dstp.reshape(IDX_ROWS, CHUNK))
    return _fin(acc, z, dis, b1.reshape(NC, HALF), W_fc,
                b_fc.reshape(1, N_CLASSES))


# speedup vs baseline: 10.1961x; 10.1961x over previous
import dataclasses
import functools

import jax
import jax.numpy as jnp
from jax import lax
from jax.experimental import pallas as pl
from jax.experimental.pallas import tpu as pltpu
from jax.experimental.pallas import tpu_sc as plsc

N_NODES = 10000
N_EDGES = 160000
D_FEAT = 256
HALF = 128
N_CLASSES = 2
NC = 2
NS = 16
L = 16

CHUNK = 128
CHUNKS_PER_TILE = 80
E_TILE = CHUNK * CHUNKS_PER_TILE
E_PAD = E_TILE * NS
IDX_ROWS = E_PAD // CHUNK
ACC_ROWS = N_NODES + 112

E_W = E_PAD // (NC * NS)
BLK_W = E_W // L
DEG_ROWS = 640

_mesh = plsc.VectorSubcoreMesh(core_axis_name="c", subcore_axis_name="s")

_sc_params = pltpu.CompilerParams()
if "needs_layout_passes" in pltpu.CompilerParams.__dataclass_fields__:
    _sc_params = dataclasses.replace(_sc_params, needs_layout_passes=False)


@functools.partial(
    pl.kernel,
    out_type=jax.ShapeDtypeStruct((NC, DEG_ROWS, L), jnp.float32),
    mesh=_mesh,
    compiler_params=_sc_params,
    scratch_types=[
        pltpu.VMEM((E_W,), jnp.int32),
        pltpu.VMEM((DEG_ROWS, L), jnp.float32),
        pltpu.VMEM((5, CHUNK), jnp.int32),
        pltpu.VMEM((40, L), jnp.float32),
        pltpu.VMEM_SHARED((DEG_ROWS, L), jnp.float32),
    ],
)
def _deg_kernel(dstp_hbm, iota_hbm, out_hbm, idx_v, hist, iota_v, zrow, deg_sh):
    c = lax.axis_index("c")
    s = lax.axis_index("s")
    w = c * NS + s

    zeros16 = jnp.zeros((L,), jnp.float32)

    @pl.loop(0, DEG_ROWS)
    def _(i):
        hist[i, :] = zeros16

    @pl.loop(0, 40)
    def _(i):
        zrow[i, :] = zeros16

    pltpu.sync_copy(zrow, deg_sh.at[pl.ds(s * 40, 40)])
    pltpu.sync_copy(iota_hbm, iota_v)
    pltpu.sync_copy(dstp_hbm.at[pl.ds(w * E_W, E_W)], idx_v)
    plsc.subcore_barrier()

    ones16 = jnp.ones((L,), jnp.float32)

    @pl.loop(0, BLK_W)
    def _(j):
        d = idx_v[pl.ds(j * L, L)]
        rows = lax.shift_right_logical(d, 4)
        cols = lax.bitwise_and(d, 15)
        plsc.addupdate_scatter(hist, [rows, cols], ones16)

    for r in range(5):
        pltpu.sync_copy(
            hist.at[pl.ds(r * CHUNK, CHUNK)],
            deg_sh.at[iota_v.at[r]],
            add=True,
        )
    plsc.subcore_barrier()
    pltpu.sync_copy(deg_sh.at[pl.ds(s * 40, 40)], out_hbm.at[c, pl.ds(s * 40, 40)])


def _mm_body(x_ref, w_ref, degt_ref, z_ref, dis_ref):
    xw = jnp.dot(x_ref[...], w_ref[...], preferred_element_type=jnp.float32)
    deg = degt_ref[:, 0:1] + degt_ref[:, 1:2] + 1.0
    dis = lax.rsqrt(deg)
    z_ref[0, :, :] = xw[:, :HALF] * dis
    z_ref[1, :, :] = xw[:, HALF:] * dis
    dis_ref[...] = dis


_ROWS_BLK = 1000


def _mm(x, w1, degt):
    return pl.pallas_call(
        _mm_body,
        grid=(N_NODES // _ROWS_BLK,),
        in_specs=[
            pl.BlockSpec((_ROWS_BLK, D_FEAT), lambda i: (i, 0)),
            pl.BlockSpec((D_FEAT, D_FEAT), lambda i: (0, 0)),
            pl.BlockSpec((_ROWS_BLK, 2), lambda i: (i, 0)),
        ],
        out_specs=[
            pl.BlockSpec((NC, _ROWS_BLK, HALF), lambda i: (0, i, 0)),
            pl.BlockSpec((_ROWS_BLK, 1), lambda i: (i, 0)),
        ],
        out_shape=[
            jax.ShapeDtypeStruct((NC, N_NODES, HALF), jnp.float32),
            jax.ShapeDtypeStruct((N_NODES, 1), jnp.float32),
        ],
    )(x, w1, degt)


@functools.partial(
    pl.kernel,
    out_type=jax.ShapeDtypeStruct((NC, N_NODES, HALF), jnp.float32),
    mesh=_mesh,
    compiler_params=_sc_params,
    scratch_types=[
        pltpu.VMEM((CHUNKS_PER_TILE, CHUNK), jnp.int32),
        pltpu.VMEM((CHUNKS_PER_TILE, CHUNK), jnp.int32),
        pltpu.VMEM((CHUNK, HALF), jnp.float32),
        pltpu.VMEM_SHARED((ACC_ROWS, HALF), jnp.float32),
    ],
)
def _agg_kernel(z_hbm, srcp_hbm, dstp_hbm, out_hbm, src_v, dst_v, buf, acc_sh):
    c = lax.axis_index("c")
    s = lax.axis_index("s")

    zeros16 = jnp.zeros((L,), jnp.float32)

    @pl.loop(0, CHUNK)
    def _(i):
        for k in range(HALF // L):
            buf[i, pl.ds(k * L, L)] = zeros16

    base = s * 632
    for off, n in ((0, 128), (128, 128), (256, 128), (384, 128), (512, 120)):
        pltpu.sync_copy(buf.at[pl.ds(0, n)], acc_sh.at[pl.ds(base + off, n)])

    pltpu.sync_copy(srcp_hbm.at[pl.ds(s * CHUNKS_PER_TILE, CHUNKS_PER_TILE)], src_v)
    pltpu.sync_copy(dstp_hbm.at[pl.ds(s * CHUNKS_PER_TILE, CHUNKS_PER_TILE)], dst_v)
    plsc.subcore_barrier()

    zc = z_hbm.at[c]

    @pl.loop(0, CHUNKS_PER_TILE)
    def _(j):
        pltpu.sync_copy(zc.at[src_v.at[j]], buf)
        pltpu.sync_copy(buf, acc_sh.at[dst_v.at[j]], add=True)

    plsc.subcore_barrier()
    rows = s * 624
    pltpu.sync_copy(
        acc_sh.at[pl.ds(rows, 624)], out_hbm.at[c, pl.ds(rows, 624)]
    )

    @pl.when(s == 0)
    def _():
        pltpu.sync_copy(
            acc_sh.at[pl.ds(9984, 16)], out_hbm.at[c, pl.ds(9984, 16)]
        )


def _fin_body(acc_ref, z_ref, dis_ref, b1_ref, wfc_ref, bfc_ref, out_ref, s_acc):
    i = pl.program_id(0)
    dis = dis_ref[...]
    h0 = jnp.maximum(dis * (acc_ref[0, :, :] + z_ref[0, :, :]) + b1_ref[0:1, :], 0.0)
    h1 = jnp.maximum(dis * (acc_ref[1, :, :] + z_ref[1, :, :]) + b1_ref[1:2, :], 0.0)
    p0 = jnp.sum(h0, axis=0, keepdims=True)
    p1 = jnp.sum(h1, axis=0, keepdims=True)

    @pl.when(i == 0)
    def _():
        s_acc[0:1, :] = p0
        s_acc[1:2, :] = p1

    @pl.when(i > 0)
    def _():
        s_acc[0:1, :] = s_acc[0:1, :] + p0
        s_acc[1:2, :] = s_acc[1:2, :] + p1

    @pl.when(i == pl.num_programs(0) - 1)
    def _():
        h = s_acc[...] * (1.0 / N_NODES)
        logits = (
            jnp.dot(h[0:1, :], wfc_ref[pl.ds(0, HALF), :],
                    preferred_element_type=jnp.float32)
            + jnp.dot(h[1:2, :], wfc_ref[pl.ds(HALF, HALF), :],
                      preferred_element_type=jnp.float32)
            + bfc_ref[...]
        )
        m = jnp.max(logits, axis=1, keepdims=True)
        lse = m + jnp.log(jnp.sum(jnp.exp(logits - m), axis=1, keepdims=True))
        out_ref[...] = logits - lse


def _fin(acc, z, dis, b1r, wfc, bfcr):
    return pl.pallas_call(
        _fin_body,
        grid=(N_NODES // _ROWS_BLK,),
        in_specs=[
            pl.BlockSpec((NC, _ROWS_BLK, HALF), lambda i: (0, i, 0)),
            pl.BlockSpec((NC, _ROWS_BLK, HALF), lambda i: (0, i, 0)),
            pl.BlockSpec((_ROWS_BLK, 1), lambda i: (i, 0)),
            pl.BlockSpec((NC, HALF), lambda i: (0, 0)),
            pl.BlockSpec((D_FEAT, N_CLASSES), lambda i: (0, 0)),
            pl.BlockSpec((1, N_CLASSES), lambda i: (0, 0)),
        ],
        out_specs=pl.BlockSpec((1, N_CLASSES), lambda i: (0, 0)),
        out_shape=jax.ShapeDtypeStruct((1, N_CLASSES), jnp.float32),
        scratch_shapes=[pltpu.VMEM((NC, HALF), jnp.float32)],
    )(acc, z, dis, b1r, wfc, bfcr)


def kernel(x, edge_index, W1, b1, W_fc, b_fc):
    src = edge_index[0]
    dst = edge_index[1]
    pad = E_PAD - N_EDGES
    srcp = jnp.concatenate([src, jnp.zeros((pad,), jnp.int32)])
    dstp = jnp.concatenate([dst, jnp.full((pad,), N_NODES, jnp.int32)])
    iota5 = jnp.arange(DEG_ROWS, dtype=jnp.int32).reshape(5, CHUNK)

    degp = _deg_kernel(dstp, iota5)
    degt = degp.reshape(NC, DEG_ROWS * L)[:, :N_NODES].T
    z, dis = _mm(x, W1, degt)
    acc = _agg_kernel(z, srcp.reshape(IDX_ROWS, CHUNK),
                      dstp.reshape(IDX_ROWS, CHUNK))
    return _fin(acc, z, dis, b1.reshape(NC, HALF), W_fc,
                b_fc.reshape(1, N_CLASSES))


# speedup vs baseline: 11.3346x; 1.1117x over previous
import dataclasses
import functools

import jax
import jax.numpy as jnp
from jax import lax
from jax.experimental import pallas as pl
from jax.experimental.pallas import tpu as pltpu
from jax.experimental.pallas import tpu_sc as plsc

N_NODES = 10000
N_EDGES = 160000
D_FEAT = 256
HALF = 128
N_CLASSES = 2
NC = 2
NS = 16
L = 16

CHUNK = 128
CHUNKS_PER_TILE = 80
E_TILE = CHUNK * CHUNKS_PER_TILE
E_PAD = E_TILE * NS
IDX_ROWS = E_PAD // CHUNK
ACC_ROWS = N_NODES + 112

E_W = E_PAD // (NC * NS)
BLK_W = E_W // L
DEG_ROWS = 640

_mesh = plsc.VectorSubcoreMesh(core_axis_name="c", subcore_axis_name="s")

_sc_params = pltpu.CompilerParams()
if "needs_layout_passes" in pltpu.CompilerParams.__dataclass_fields__:
    _sc_params = dataclasses.replace(_sc_params, needs_layout_passes=False)


@functools.partial(
    pl.kernel,
    out_type=jax.ShapeDtypeStruct((NC, DEG_ROWS, L), jnp.float32),
    mesh=_mesh,
    compiler_params=_sc_params,
    scratch_types=[
        pltpu.VMEM((E_W,), jnp.int32),
        pltpu.VMEM((DEG_ROWS, L), jnp.float32),
        pltpu.VMEM((5, CHUNK), jnp.int32),
        pltpu.VMEM((40, L), jnp.float32),
        pltpu.VMEM_SHARED((DEG_ROWS, L), jnp.float32),
    ],
)
def _deg_kernel(dstp_hbm, iota_hbm, out_hbm, idx_v, hist, iota_v, zrow, deg_sh):
    c = lax.axis_index("c")
    s = lax.axis_index("s")
    w = c * NS + s

    zeros16 = jnp.zeros((L,), jnp.float32)

    @pl.loop(0, DEG_ROWS)
    def _(i):
        hist[i, :] = zeros16

    @pl.loop(0, 40)
    def _(i):
        zrow[i, :] = zeros16

    pltpu.sync_copy(zrow, deg_sh.at[pl.ds(s * 40, 40)])
    pltpu.sync_copy(iota_hbm, iota_v)
    pltpu.sync_copy(dstp_hbm.at[pl.ds(w * E_W, E_W)], idx_v)
    plsc.subcore_barrier()

    ones16 = jnp.ones((L,), jnp.float32)

    @pl.loop(0, BLK_W)
    def _(j):
        d = idx_v[pl.ds(j * L, L)]
        rows = lax.shift_right_logical(d, 4)
        cols = lax.bitwise_and(d, 15)
        plsc.addupdate_scatter(hist, [rows, cols], ones16)

    for r in range(5):
        pltpu.sync_copy(
            hist.at[pl.ds(r * CHUNK, CHUNK)],
            deg_sh.at[iota_v.at[r]],
            add=True,
        )
    plsc.subcore_barrier()
    pltpu.sync_copy(deg_sh.at[pl.ds(s * 40, 40)], out_hbm.at[c, pl.ds(s * 40, 40)])


def _mm_body(x_ref, w_ref, degt_ref, z_ref, dis_ref):
    xw = jnp.dot(x_ref[...], w_ref[...], preferred_element_type=jnp.float32)
    deg = degt_ref[:, 0:1] + degt_ref[:, 1:2] + 1.0
    dis = lax.rsqrt(deg)
    z_ref[0, :, :] = xw[:, :HALF] * dis
    z_ref[1, :, :] = xw[:, HALF:] * dis
    dis_ref[...] = dis


_ROWS_BLK = 1000


def _mm(x, w1, degt):
    return pl.pallas_call(
        _mm_body,
        grid=(N_NODES // _ROWS_BLK,),
        in_specs=[
            pl.BlockSpec((_ROWS_BLK, D_FEAT), lambda i: (i, 0)),
            pl.BlockSpec((D_FEAT, D_FEAT), lambda i: (0, 0)),
            pl.BlockSpec((_ROWS_BLK, 2), lambda i: (i, 0)),
        ],
        out_specs=[
            pl.BlockSpec((NC, _ROWS_BLK, HALF), lambda i: (0, i, 0)),
            pl.BlockSpec((_ROWS_BLK, 1), lambda i: (i, 0)),
        ],
        out_shape=[
            jax.ShapeDtypeStruct((NC, N_NODES, HALF), jnp.float32),
            jax.ShapeDtypeStruct((N_NODES, 1), jnp.float32),
        ],
    )(x, w1, degt)


@functools.partial(
    pl.kernel,
    out_type=jax.ShapeDtypeStruct((NC, N_NODES, HALF), jnp.float32),
    mesh=_mesh,
    compiler_params=_sc_params,
    scratch_types=[
        pltpu.VMEM((CHUNKS_PER_TILE // 2, CHUNK), jnp.int32),
        pltpu.VMEM((CHUNKS_PER_TILE // 2, CHUNK), jnp.int32),
        pltpu.VMEM((CHUNK, HALF), jnp.float32),
        pltpu.VMEM((CHUNK, HALF), jnp.float32),
        pltpu.VMEM_SHARED((ACC_ROWS, HALF), jnp.float32),
        pltpu.SemaphoreType.DMA,
        pltpu.SemaphoreType.DMA,
        pltpu.SemaphoreType.DMA,
        pltpu.SemaphoreType.DMA,
    ],
)
def _agg_kernel(z_hbm, srcp_hbm, dstp_hbm, out_hbm, src_v, dst_v, buf0, buf1,
                acc_sh, sg0, sg1, ss0, ss1):
    c = lax.axis_index("c")
    s = lax.axis_index("s")

    zeros16 = jnp.zeros((L,), jnp.float32)

    @pl.loop(0, CHUNK)
    def _(i):
        for k in range(HALF // L):
            buf0[i, pl.ds(k * L, L)] = zeros16

    base = s * 632
    for off, n in ((0, 128), (128, 128), (256, 128), (384, 128), (512, 120)):
        pltpu.sync_copy(buf0.at[pl.ds(0, n)], acc_sh.at[pl.ds(base + off, n)])

    plsc.subcore_barrier()

    zc = z_hbm.at[c]
    npass = CHUNKS_PER_TILE // 2

    for p in range(2):
        pltpu.sync_copy(
            srcp_hbm.at[pl.ds(s * CHUNKS_PER_TILE + p * npass, npass)], src_v)
        pltpu.sync_copy(
            dstp_hbm.at[pl.ds(s * CHUNKS_PER_TILE + p * npass, npass)], dst_v)

        pltpu.async_copy(zc.at[src_v.at[0]], buf0, sg0)

        @pl.loop(0, npass, step=2)
        def _(j):
            pltpu.make_async_copy(zc.at[src_v.at[j]], buf0, sg0).wait()
            pltpu.async_copy(buf0, acc_sh.at[dst_v.at[j]], ss0, add=True)

            @pl.when(j > 0)
            def _():
                pltpu.make_async_copy(buf1, acc_sh.at[dst_v.at[j]], ss1).wait()

            pltpu.async_copy(zc.at[src_v.at[j + 1]], buf1, sg1)
            pltpu.make_async_copy(zc.at[src_v.at[j + 1]], buf1, sg1).wait()
            pltpu.async_copy(buf1, acc_sh.at[dst_v.at[j + 1]], ss1, add=True)
            pltpu.make_async_copy(buf0, acc_sh.at[dst_v.at[j]], ss0).wait()

            @pl.when(j + 2 < npass)
            def _():
                pltpu.async_copy(zc.at[src_v.at[j + 2]], buf0, sg0)

        pltpu.make_async_copy(buf1, acc_sh.at[dst_v.at[0]], ss1).wait()

    plsc.subcore_barrier()
    rows = s * 624
    pltpu.sync_copy(
        acc_sh.at[pl.ds(rows, 624)], out_hbm.at[c, pl.ds(rows, 624)]
    )

    @pl.when(s == 0)
    def _():
        pltpu.sync_copy(
            acc_sh.at[pl.ds(9984, 16)], out_hbm.at[c, pl.ds(9984, 16)]
        )


def _fin_body(acc_ref, z_ref, dis_ref, b1_ref, wfc_ref, bfc_ref, out_ref, s_acc):
    i = pl.program_id(0)
    dis = dis_ref[...]
    h0 = jnp.maximum(dis * (acc_ref[0, :, :] + z_ref[0, :, :]) + b1_ref[0:1, :], 0.0)
    h1 = jnp.maximum(dis * (acc_ref[1, :, :] + z_ref[1, :, :]) + b1_ref[1:2, :], 0.0)
    p0 = jnp.sum(h0, axis=0, keepdims=True)
    p1 = jnp.sum(h1, axis=0, keepdims=True)

    @pl.when(i == 0)
    def _():
        s_acc[0:1, :] = p0
        s_acc[1:2, :] = p1

    @pl.when(i > 0)
    def _():
        s_acc[0:1, :] = s_acc[0:1, :] + p0
        s_acc[1:2, :] = s_acc[1:2, :] + p1

    @pl.when(i == pl.num_programs(0) - 1)
    def _():
        h = s_acc[...] * (1.0 / N_NODES)
        logits = (
            jnp.dot(h[0:1, :], wfc_ref[pl.ds(0, HALF), :],
                    preferred_element_type=jnp.float32)
            + jnp.dot(h[1:2, :], wfc_ref[pl.ds(HALF, HALF), :],
                      preferred_element_type=jnp.float32)
            + bfc_ref[...]
        )
        m = jnp.max(logits, axis=1, keepdims=True)
        lse = m + jnp.log(jnp.sum(jnp.exp(logits - m), axis=1, keepdims=True))
        out_ref[...] = logits - lse


def _fin(acc, z, dis, b1r, wfc, bfcr):
    return pl.pallas_call(
        _fin_body,
        grid=(N_NODES // _ROWS_BLK,),
        in_specs=[
            pl.BlockSpec((NC, _ROWS_BLK, HALF), lambda i: (0, i, 0)),
            pl.BlockSpec((NC, _ROWS_BLK, HALF), lambda i: (0, i, 0)),
            pl.BlockSpec((_ROWS_BLK, 1), lambda i: (i, 0)),
            pl.BlockSpec((NC, HALF), lambda i: (0, 0)),
            pl.BlockSpec((D_FEAT, N_CLASSES), lambda i: (0, 0)),
            pl.BlockSpec((1, N_CLASSES), lambda i: (0, 0)),
        ],
        out_specs=pl.BlockSpec((1, N_CLASSES), lambda i: (0, 0)),
        out_shape=jax.ShapeDtypeStruct((1, N_CLASSES), jnp.float32),
        scratch_shapes=[pltpu.VMEM((NC, HALF), jnp.float32)],
    )(acc, z, dis, b1r, wfc, bfcr)


def kernel(x, edge_index, W1, b1, W_fc, b_fc):
    src = edge_index[0]
    dst = edge_index[1]
    pad = E_PAD - N_EDGES
    srcp = jnp.concatenate([src, jnp.zeros((pad,), jnp.int32)])
    dstp = jnp.concatenate([dst, jnp.full((pad,), N_NODES, jnp.int32)])
    iota5 = jnp.arange(DEG_ROWS, dtype=jnp.int32).reshape(5, CHUNK)

    degp = _deg_kernel(dstp, iota5)
    degt = degp.reshape(NC, DEG_ROWS * L)[:, :N_NODES].T
    z, dis = _mm(x, W1, degt)
    acc = _agg_kernel(z, srcp.reshape(IDX_ROWS, CHUNK),
                      dstp.reshape(IDX_ROWS, CHUNK))
    return _fin(acc, z, dis, b1.reshape(NC, HALF), W_fc,
                b_fc.reshape(1, N_CLASSES))
